# baseline (speedup 1.0000x reference)
import dataclasses
import functools

import jax
import jax.numpy as jnp
from jax import lax
from jax.experimental import pallas as pl
from jax.experimental.pallas import tpu as pltpu
from jax.experimental.pallas import tpu_sc as plsc

N = 10000
E = 160000
F = 256

NC = 2
NS = 16
NT = NC * NS
L = 16
CPT = F // NT

NPAD = 10240
WIN = 1024
NW = 158
EPAD = WIN * NW
EPT = EPAD // NT
GRP = EPT // L

_mesh = plsc.VectorSubcoreMesh(core_axis_name="c", subcore_axis_name="s")

_sc_params = pltpu.CompilerParams()
if "needs_layout_passes" in pltpu.CompilerParams.__dataclass_fields__:
    _sc_params = dataclasses.replace(_sc_params, needs_layout_passes=False)
_sc_main_params = dataclasses.replace(_sc_params, use_tc_tiling_on_sc=False)


@functools.partial(
    pl.kernel,
    mesh=_mesh,
    compiler_params=_sc_main_params,
    out_type=jax.ShapeDtypeStruct((NT, NPAD), jnp.float32),
    scratch_types=[
        pltpu.VMEM((EPT,), jnp.int32),
        pltpu.VMEM((EPT,), jnp.int32),
        pltpu.VMEM((EPT,), jnp.float32),
        pltpu.VMEM((NPAD,), jnp.float32),
    ],
)
def _sc_degree(src_hbm, dst_hbm, attr_hbm, deg_out, src_v, dst_v, attr_v, acc):
    cid = lax.axis_index("c")
    sid = lax.axis_index("s")
    tile = cid * NS + sid
    base = tile * EPT

    pltpu.sync_copy(src_hbm.at[pl.ds(base, EPT)], src_v)
    pltpu.sync_copy(dst_hbm.at[pl.ds(base, EPT)], dst_v)
    pltpu.sync_copy(attr_hbm.at[pl.ds(base, EPT)], attr_v)

    zero = jnp.zeros((L,), jnp.float32)

    @pl.loop(0, NPAD // (8 * L))
    def _(i):
        for u in range(8):
            acc[pl.ds((i * 8 + u) * L, L)] = zero

    @pl.loop(0, GRP)
    def _(g):
        sv = src_v[pl.ds(g * L, L)]
        dv = dst_v[pl.ds(g * L, L)]
        wv = attr_v[pl.ds(g * L, L)]
        w = jnp.where(sv == dv, 0.0, wv)
        plsc.addupdate_scatter(acc, [sv], w)

    pltpu.sync_copy(acc, deg_out.at[tile])


def _dis_body(deg_ref, dis_ref):
    d = jnp.sum(deg_ref[...], axis=0)
    dis_ref[...] = jnp.where(d > 0, lax.rsqrt(jnp.where(d > 0, d, 1.0)), 0.0)


def _tc_dis(deg_parts):
    return pl.pallas_call(
        _dis_body,
        out_shape=jax.ShapeDtypeStruct((80, 128), jnp.float32),
    )(deg_parts)


@functools.partial(
    pl.kernel,
    mesh=_mesh,
    compiler_params=_sc_main_params,
    out_type=[jax.ShapeDtypeStruct((EPAD,), jnp.float32),
              jax.ShapeDtypeStruct((EPAD,), jnp.int32)],
    scratch_types=[
        pltpu.VMEM((EPT,), jnp.int32),
        pltpu.VMEM((EPT,), jnp.int32),
        pltpu.VMEM((EPT,), jnp.float32),
        pltpu.VMEM((EPT,), jnp.float32),
        pltpu.VMEM((EPT,), jnp.int32),
        pltpu.VMEM((NPAD,), jnp.float32),
    ],
)
def _sc_lam(src_hbm, dst_hbm, attr_hbm, dis_hbm, lam_out, dv8_out,
            src_v, dst_v, attr_v, lam_v, dv8_v, dis_v):
    cid = lax.axis_index("c")
    sid = lax.axis_index("s")
    base = (cid * NS + sid) * EPT

    pltpu.sync_copy(src_hbm.at[pl.ds(base, EPT)], src_v)
    pltpu.sync_copy(dst_hbm.at[pl.ds(base, EPT)], dst_v)
    pltpu.sync_copy(attr_hbm.at[pl.ds(base, EPT)], attr_v)
    pltpu.sync_copy(dis_hbm, dis_v)

    @pl.loop(0, GRP)
    def _(g):
        sv = src_v[pl.ds(g * L, L)]
        dv = dst_v[pl.ds(g * L, L)]
        wv = attr_v[pl.ds(g * L, L)]
        w = jnp.where(sv == dv, 0.0, wv)
        dis_s = plsc.load_gather(dis_v, [sv])
        dis_d = plsc.load_gather(dis_v, [dv])
        lam_v[pl.ds(g * L, L)] = -(dis_s * w * dis_d)
        dv8_v[pl.ds(g * L, L)] = dv << 3

    pltpu.sync_copy(lam_v, lam_out.at[pl.ds(base, EPT)])
    pltpu.sync_copy(dv8_v, dv8_out.at[pl.ds(base, EPT)])


@functools.partial(
    pl.kernel,
    mesh=_mesh,
    compiler_params=_sc_main_params,
    out_type=jax.ShapeDtypeStruct((NT, NPAD * CPT), jnp.float32),
    scratch_types=[
        pltpu.VMEM((NPAD * CPT,), jnp.float32),
        pltpu.VMEM((8, 128), jnp.int32),
        pltpu.VMEM((8, 128), jnp.int32),
        pltpu.VMEM((WIN,), jnp.int32),
        pltpu.VMEM((WIN,), jnp.int32),
        pltpu.VMEM((WIN,), jnp.float32),
        pltpu.VMEM((WIN,), jnp.float32),
        pltpu.VMEM((WIN, CPT), jnp.float32),
        pltpu.VMEM((WIN, CPT), jnp.float32),
        pltpu.SemaphoreType.DMA,
        pltpu.SemaphoreType.DMA,
        pltpu.SemaphoreType.DMA,
        pltpu.SemaphoreType.DMA,
    ],
)
def _sc_main(xt_hbm, src2d_hbm, dst_hbm, lam_hbm, tx_out,
             acc, sw0, sw1, dw0, dw1, lw0, lw1, gb0, gb1,
             ms0, ms1, gs0, gs1):
    cid = lax.axis_index("c")
    sid = lax.axis_index("s")
    tile = cid * NS + sid
    roff = tile * N

    sw = (sw0, sw1)
    dw = (dw0, dw1)
    lw = (lw0, lw1)
    gb = (gb0, gb1)
    ms = (ms0, ms1)
    gs = (gs0, gs1)

    zero = jnp.zeros((L,), jnp.float32)
    iota = lax.iota(jnp.int32, L)
    io8 = iota >> 3
    col8 = iota & 7

    @pl.loop(0, NPAD * CPT // (8 * L))
    def _(i):
        for u in range(8):
            acc[pl.ds((i * 8 + u) * L, L)] = zero

    def issue_meta(w, p):
        pltpu.async_copy(src2d_hbm.at[pl.ds(w * 8, 8)], sw[p], ms[p])
        pltpu.async_copy(dst_hbm.at[pl.ds(w * WIN, WIN)], dw[p], ms[p])
        pltpu.async_copy(lam_hbm.at[pl.ds(w * WIN, WIN)], lw[p], ms[p])

    def wait_meta(w, p):
        pltpu.make_async_copy(src2d_hbm.at[pl.ds(w * 8, 8)], sw[p], ms[p]).wait()
        pltpu.make_async_copy(dst_hbm.at[pl.ds(w * WIN, WIN)], dw[p], ms[p]).wait()
        pltpu.make_async_copy(lam_hbm.at[pl.ds(w * WIN, WIN)], lw[p], ms[p]).wait()

    def issue_gathers(p):
        for j in range(8):
            for k in range(8):
                sl = (j, pl.ds(k * L, L))
                sw[p][sl] = sw[p][sl] + roff
        for j in range(8):
            pltpu.async_copy(xt_hbm.at[sw[p].at[j]],
                             gb[p].at[pl.ds(j * 128, 128)], gs[p])

    def wait_gathers(p):
        pltpu.make_async_copy(xt_hbm.at[pl.ds(0, WIN)], gb[p], gs[p]).wait()

    issue_meta(0, 0)
    wait_meta(0, 0)
    issue_gathers(0)
    issue_meta(1, 1)

    @pl.loop(0, NW, step=2)
    def _(i):
        for par in (0, 1):
            w = i + par
            q = 1 - par

            @pl.when(w + 1 < NW)
            def _():
                wait_meta(w + 1, q)
                issue_gathers(q)

            wait_gathers(par)
            gflat = gb[par]
            dvw = dw[par]
            lmw = lw[par]

            @plsc.parallel_loop(0, WIN // 2, unroll=8)
            def _(p):
                pvec = jnp.full((L,), 2 * p, jnp.int32) + io8
                g = plsc.load_gather(gflat, [pvec, col8])
                lm = plsc.load_gather(lmw, [pvec])
                dv8 = plsc.load_gather(dvw, [pvec])
                plsc.addupdate_scatter(acc, [dv8 | col8], g * lm)

            @pl.when(w + 2 < NW)
            def _():
                issue_meta(w + 2, par)

    pltpu.sync_copy(acc, tx_out.at[tile])


def _m0_body(x_ref, w0t_ref, b_ref, o_ref):
    o_ref[...] = jnp.dot(x_ref[...], w0t_ref[...],
                         precision=lax.Precision.HIGHEST,
                         preferred_element_type=jnp.float32) + b_ref[...]


def _tc_m0(x, w0t, b2):
    blk = 1000
    return pl.pallas_call(
        _m0_body,
        grid=(N // blk,),
        in_specs=[
            pl.BlockSpec((blk, F), lambda i: (i, 0)),
            pl.BlockSpec((F, F), lambda i: (0, 0)),
            pl.BlockSpec((1, F), lambda i: (0, 0)),
        ],
        out_specs=pl.BlockSpec((blk, F), lambda i: (i, 0)),
        out_shape=jax.ShapeDtypeStruct((N, F), jnp.float32),
    )(x, w0t, b2)


def _out_body(m0_ref, tx_ref, w1t_ref, o_ref):
    acc = m0_ref[...] + jnp.dot(tx_ref[...], w1t_ref[...],
                                precision=lax.Precision.HIGHEST,
                                preferred_element_type=jnp.float32)
    o_ref[...] = jnp.where(acc >= 0, acc, 0.01 * acc)


def _tc_out(m0, tx1, w1t):
    blk = 1000
    return pl.pallas_call(
        _out_body,
        grid=(N // blk,),
        in_specs=[
            pl.BlockSpec((blk, F), lambda i: (i, 0)),
            pl.BlockSpec((blk, F), lambda i: (i, 0)),
            pl.BlockSpec((F, F), lambda i: (0, 0)),
        ],
        out_specs=pl.BlockSpec((blk, F), lambda i: (i, 0)),
        out_shape=jax.ShapeDtypeStruct((N, F), jnp.float32),
    )(m0, tx1, w1t)


def kernel(x, edge_index, edge_attr, nroi, W0, W1, b):
    del nroi
    src = edge_index[0].astype(jnp.int32)
    dst = edge_index[1].astype(jnp.int32)
    pad = EPAD - E
    spread = (jnp.arange(pad, dtype=jnp.int32) * 61) % N
    src_p = jnp.concatenate([src, spread])
    dst_p = jnp.concatenate([dst, spread])
    attr_p = jnp.concatenate([edge_attr, jnp.zeros((pad,), jnp.float32)])

    deg_parts = _sc_degree(src_p, dst_p, attr_p)
    dis = _tc_dis(deg_parts.reshape(NT, 80, 128)).reshape(NPAD)
    lam, dv8 = _sc_lam(src_p, dst_p, attr_p, dis)

    xt = x.reshape(N, NT, CPT).transpose(1, 0, 2).reshape(NT * N, CPT)
    src2d = src_p.reshape(NW * 8, 128)

    tx_parts = _sc_main(xt, src2d, dv8, lam)

    tx1 = tx_parts.reshape(NT, NPAD, CPT).transpose(1, 0, 2).reshape(NPAD, F)[:N]

    m0 = _tc_m0(x, W0.T, b.reshape(1, F))
    return _tc_out(m0, tx1, W1.T)


# speedup vs baseline: 1.0168x; 1.0168x over previous
import dataclasses
import functools

import jax
import jax.numpy as jnp
from jax import lax
from jax.experimental import pallas as pl
from jax.experimental.pallas import tpu as pltpu
from jax.experimental.pallas import tpu_sc as plsc

N = 10000
E = 160000
F = 256

NC = 2
NS = 16
NT = NC * NS
L = 16
CPT = F // NT

NPAD = 10240
WIN = 1024
NW = 158
EPAD = WIN * NW
EPT = EPAD // NT
GRP = EPT // L

_mesh = plsc.VectorSubcoreMesh(core_axis_name="c", subcore_axis_name="s")

_sc_params = pltpu.CompilerParams()
if "needs_layout_passes" in pltpu.CompilerParams.__dataclass_fields__:
    _sc_params = dataclasses.replace(_sc_params, needs_layout_passes=False)
_sc_main_params = dataclasses.replace(_sc_params, use_tc_tiling_on_sc=False)


EPSC = EPAD // NS
GRP1 = EPSC // L
SNOD = NPAD // NS


@functools.partial(
    pl.kernel,
    mesh=_mesh,
    compiler_params=_sc_main_params,
    out_type=[jax.ShapeDtypeStruct((EPAD,), jnp.float32),
              jax.ShapeDtypeStruct((EPAD,), jnp.int32)],
    scratch_types=[
        pltpu.VMEM((EPSC,), jnp.int32),
        pltpu.VMEM((EPSC,), jnp.int32),
        pltpu.VMEM((EPSC,), jnp.float32),
        pltpu.VMEM((EPT,), jnp.float32),
        pltpu.VMEM((EPT,), jnp.int32),
        pltpu.VMEM((NPAD,), jnp.float32),
        pltpu.VMEM((SNOD,), jnp.float32),
        pltpu.VMEM((SNOD,), jnp.float32),
        pltpu.VMEM_SHARED((NS, NPAD), jnp.float32),
        pltpu.VMEM_SHARED((NPAD,), jnp.float32),
    ],
)
def _sc_prep(src_hbm, dst_hbm, attr_hbm, lam_out, dv8_out,
             src_v, dst_v, attr_v, lam_v, dv8_v, deg_v, rsum, rin,
             deg_sh, dis_sh):
    cid = lax.axis_index("c")
    sid = lax.axis_index("s")
    zero = jnp.zeros((L,), jnp.float32)

    base1 = sid * EPSC
    pltpu.sync_copy(src_hbm.at[pl.ds(base1, EPSC)], src_v)
    pltpu.sync_copy(dst_hbm.at[pl.ds(base1, EPSC)], dst_v)
    pltpu.sync_copy(attr_hbm.at[pl.ds(base1, EPSC)], attr_v)

    @pl.loop(0, NPAD // (8 * L))
    def _(i):
        for u in range(8):
            deg_v[pl.ds((i * 8 + u) * L, L)] = zero

    @pl.loop(0, GRP1)
    def _(g):
        sv = src_v[pl.ds(g * L, L)]
        dv = dst_v[pl.ds(g * L, L)]
        wv = attr_v[pl.ds(g * L, L)]
        w = jnp.where(sv == dv, 0.0, wv)
        plsc.addupdate_scatter(deg_v, [sv], w)

    pltpu.sync_copy(deg_v, deg_sh.at[sid])
    plsc.subcore_barrier()

    nb = sid * SNOD
    pltpu.sync_copy(deg_sh.at[0, pl.ds(nb, SNOD)], rsum)
    for k in range(1, NS):
        pltpu.sync_copy(deg_sh.at[k, pl.ds(nb, SNOD)], rin)

        @pl.loop(0, SNOD // L)
        def _(i):
            rsum[pl.ds(i * L, L)] = rsum[pl.ds(i * L, L)] + rin[pl.ds(i * L, L)]

    magic = jnp.full((L,), 0x5F3759DF, jnp.int32)

    @pl.loop(0, SNOD // L)
    def _(i):
        d = rsum[pl.ds(i * L, L)]
        y = plsc.bitcast(magic - (plsc.bitcast(d, jnp.int32) >> 1),
                         jnp.float32)
        for _ in range(3):
            y = y * (1.5 - 0.5 * d * y * y)
        rsum[pl.ds(i * L, L)] = jnp.where(d > 0, y, 0.0)

    pltpu.sync_copy(rsum, dis_sh.at[pl.ds(nb, SNOD)])
    plsc.subcore_barrier()
    pltpu.sync_copy(dis_sh, deg_v)

    base3 = (cid * NS + sid) * EPT
    pltpu.sync_copy(src_hbm.at[pl.ds(base3, EPT)], src_v.at[pl.ds(0, EPT)])
    pltpu.sync_copy(dst_hbm.at[pl.ds(base3, EPT)], dst_v.at[pl.ds(0, EPT)])
    pltpu.sync_copy(attr_hbm.at[pl.ds(base3, EPT)], attr_v.at[pl.ds(0, EPT)])

    @pl.loop(0, GRP)
    def _(g):
        sv = src_v[pl.ds(g * L, L)]
        dv = dst_v[pl.ds(g * L, L)]
        wv = attr_v[pl.ds(g * L, L)]
        w = jnp.where(sv == dv, 0.0, wv)
        dis_s = plsc.load_gather(deg_v, [sv])
        dis_d = plsc.load_gather(deg_v, [dv])
        lam_v[pl.ds(g * L, L)] = -(dis_s * w * dis_d)
        dv8_v[pl.ds(g * L, L)] = dv << 3

    pltpu.sync_copy(lam_v, lam_out.at[pl.ds(base3, EPT)])
    pltpu.sync_copy(dv8_v, dv8_out.at[pl.ds(base3, EPT)])


@functools.partial(
    pl.kernel,
    mesh=_mesh,
    compiler_params=_sc_main_params,
    out_type=jax.ShapeDtypeStruct((NT, NPAD * CPT), jnp.float32),
    scratch_types=[
        pltpu.VMEM((NPAD * CPT,), jnp.float32),
        pltpu.VMEM((8, 128), jnp.int32),
        pltpu.VMEM((8, 128), jnp.int32),
        pltpu.VMEM((WIN,), jnp.int32),
        pltpu.VMEM((WIN,), jnp.int32),
        pltpu.VMEM((WIN,), jnp.float32),
        pltpu.VMEM((WIN,), jnp.float32),
        pltpu.VMEM((WIN, CPT), jnp.float32),
        pltpu.VMEM((WIN, CPT), jnp.float32),
        pltpu.SemaphoreType.DMA,
        pltpu.SemaphoreType.DMA,
        pltpu.SemaphoreType.DMA,
        pltpu.SemaphoreType.DMA,
    ],
)
def _sc_main(xt_hbm, src2d_hbm, dst_hbm, lam_hbm, tx_out,
             acc, sw0, sw1, dw0, dw1, lw0, lw1, gb0, gb1,
             ms0, ms1, gs0, gs1):
    cid = lax.axis_index("c")
    sid = lax.axis_index("s")
    tile = cid * NS + sid
    roff = tile * N

    sw = (sw0, sw1)
    dw = (dw0, dw1)
    lw = (lw0, lw1)
    gb = (gb0, gb1)
    ms = (ms0, ms1)
    gs = (gs0, gs1)

    zero = jnp.zeros((L,), jnp.float32)
    iota = lax.iota(jnp.int32, L)
    io8 = iota >> 3
    col8 = iota & 7

    @pl.loop(0, NPAD * CPT // (8 * L))
    def _(i):
        for u in range(8):
            acc[pl.ds((i * 8 + u) * L, L)] = zero

    def issue_meta(w, p):
        pltpu.async_copy(src2d_hbm.at[pl.ds(w * 8, 8)], sw[p], ms[p])
        pltpu.async_copy(dst_hbm.at[pl.ds(w * WIN, WIN)], dw[p], ms[p])
        pltpu.async_copy(lam_hbm.at[pl.ds(w * WIN, WIN)], lw[p], ms[p])

    def wait_meta(w, p):
        pltpu.make_async_copy(src2d_hbm.at[pl.ds(w * 8, 8)], sw[p], ms[p]).wait()
        pltpu.make_async_copy(dst_hbm.at[pl.ds(w * WIN, WIN)], dw[p], ms[p]).wait()
        pltpu.make_async_copy(lam_hbm.at[pl.ds(w * WIN, WIN)], lw[p], ms[p]).wait()

    def issue_gathers(p):
        for j in range(8):
            for k in range(8):
                sl = (j, pl.ds(k * L, L))
                sw[p][sl] = sw[p][sl] + roff
        for j in range(8):
            pltpu.async_copy(xt_hbm.at[sw[p].at[j]],
                             gb[p].at[pl.ds(j * 128, 128)], gs[p])

    def wait_gathers(p):
        pltpu.make_async_copy(xt_hbm.at[pl.ds(0, WIN)], gb[p], gs[p]).wait()

    issue_meta(0, 0)
    wait_meta(0, 0)
    issue_gathers(0)
    issue_meta(1, 1)

    @pl.loop(0, NW, step=2)
    def _(i):
        for par in (0, 1):
            w = i + par
            q = 1 - par

            @pl.when(w + 1 < NW)
            def _():
                wait_meta(w + 1, q)
                issue_gathers(q)

            wait_gathers(par)
            gflat = gb[par]
            dvw = dw[par]
            lmw = lw[par]

            @plsc.parallel_loop(0, WIN // 2, unroll=8)
            def _(p):
                pvec = jnp.full((L,), 2 * p, jnp.int32) + io8
                g = plsc.load_gather(gflat, [pvec, col8])
                lm = plsc.load_gather(lmw, [pvec])
                dv8 = plsc.load_gather(dvw, [pvec])
                plsc.addupdate_scatter(acc, [dv8 | col8], g * lm)

            @pl.when(w + 2 < NW)
            def _():
                issue_meta(w + 2, par)

    pltpu.sync_copy(acc, tx_out.at[tile])


def _m0_body(x_ref, w0t_ref, b_ref, o_ref):
    o_ref[...] = jnp.dot(x_ref[...], w0t_ref[...],
                         precision=lax.Precision.HIGHEST,
                         preferred_element_type=jnp.float32) + b_ref[...]


def _tc_m0(x, w0t, b2):
    blk = 1000
    return pl.pallas_call(
        _m0_body,
        grid=(N // blk,),
        in_specs=[
            pl.BlockSpec((blk, F), lambda i: (i, 0)),
            pl.BlockSpec((F, F), lambda i: (0, 0)),
            pl.BlockSpec((1, F), lambda i: (0, 0)),
        ],
        out_specs=pl.BlockSpec((blk, F), lambda i: (i, 0)),
        out_shape=jax.ShapeDtypeStruct((N, F), jnp.float32),
    )(x, w0t, b2)


def _out_body(m0_ref, tx_ref, w1t_ref, o_ref):
    acc = m0_ref[...] + jnp.dot(tx_ref[...], w1t_ref[...],
                                precision=lax.Precision.HIGHEST,
                                preferred_element_type=jnp.float32)
    o_ref[...] = jnp.where(acc >= 0, acc, 0.01 * acc)


def _tc_out(m0, tx1, w1t):
    blk = 1000
    return pl.pallas_call(
        _out_body,
        grid=(N // blk,),
        in_specs=[
            pl.BlockSpec((blk, F), lambda i: (i, 0)),
            pl.BlockSpec((blk, F), lambda i: (i, 0)),
            pl.BlockSpec((F, F), lambda i: (0, 0)),
        ],
        out_specs=pl.BlockSpec((blk, F), lambda i: (i, 0)),
        out_shape=jax.ShapeDtypeStruct((N, F), jnp.float32),
    )(m0, tx1, w1t)


def kernel(x, edge_index, edge_attr, nroi, W0, W1, b):
    del nroi
    src = edge_index[0].astype(jnp.int32)
    dst = edge_index[1].astype(jnp.int32)
    pad = EPAD - E
    spread = (jnp.arange(pad, dtype=jnp.int32) * 61) % N
    src_p = jnp.concatenate([src, spread])
    dst_p = jnp.concatenate([dst, spread])
    attr_p = jnp.concatenate([edge_attr, jnp.zeros((pad,), jnp.float32)])

    lam, dv8 = _sc_prep(src_p, dst_p, attr_p)

    xt = x.reshape(N, NT, CPT).transpose(1, 0, 2).reshape(NT * N, CPT)
    src2d = src_p.reshape(NW * 8, 128)

    tx_parts = _sc_main(xt, src2d, dv8, lam)

    tx1 = tx_parts.reshape(NT, NPAD, CPT).transpose(1, 0, 2).reshape(NPAD, F)[:N]

    m0 = _tc_m0(x, W0.T, b.reshape(1, F))
    return _tc_out(m0, tx1, W1.T)
